# baseline (speedup 1.0000x reference)
import functools

import jax
import jax.numpy as jnp
from jax.experimental import pallas as pl
from jax.experimental.pallas import tpu as pltpu

EPS = 1e-7
_CHUNK = 1024
_UNROLL = 2


def _src_idx(out_size, in_size):
    scale = in_size / out_size
    s = (jnp.arange(out_size, dtype=jnp.float32) + 0.5) * scale - 0.5
    s = jnp.maximum(s, 0.0)
    i0 = jnp.floor(s).astype(jnp.int32)
    i1 = jnp.minimum(i0 + 1, in_size - 1)
    lam = s - i0.astype(jnp.float32)
    return i0, i1, lam


def _resize(x, out_h, out_w):
    H, W = x.shape[-2:]
    y0, y1, ly = _src_idx(out_h, H)
    x0, x1, lx = _src_idx(out_w, W)
    r = x[:, :, y0, :] * (1.0 - ly)[None, None, :, None] + x[:, :, y1, :] * ly[None, None, :, None]
    return r[:, :, :, x0] * (1.0 - lx)[None, None, None, :] + r[:, :, :, x1] * lx[None, None, None, :]


def _corner_tables(flow, nbands, band_n, npad):
    _, _, H, W = flow.shape
    N = H * W
    dump = jnp.int32(band_n)
    fx = jnp.arange(W, dtype=jnp.float32)[None, None, :] + flow[:, 0]
    fy = jnp.arange(H, dtype=jnp.float32)[None, :, None] + flow[:, 1]
    fx = jnp.clip(fx, -4.0, W + 4.0)
    fy = jnp.clip(fy, -4.0, H + 4.0)
    x0f = jnp.floor(fx)
    y0f = jnp.floor(fy)
    wx1 = fx - x0f
    wy1 = fy - y0f
    x0 = x0f.astype(jnp.int32)
    y0 = y0f.astype(jnp.int32)
    idx_bands = []
    wgt_bands = []
    for b in range(nbands):
        lo = b * band_n
        idx_c = []
        wgt_c = []
        for dy in (0, 1):
            for dx in (0, 1):
                xk = x0 + dx
                yk = y0 + dy
                wk = (wx1 if dx else 1.0 - wx1) * (wy1 if dy else 1.0 - wy1)
                local = yk * W + xk - lo
                ok = (xk >= 0) & (xk < W) & (yk >= 0) & (yk < H) \
                    & (local >= 0) & (local < band_n)
                idx_c.append(jnp.where(ok, local, dump).reshape(4, N))
                wgt_c.append(jnp.where(ok, wk, 0.0).reshape(4, N))
        idx_bands.append(jnp.stack(idx_c, axis=-1))
        wgt_bands.append(jnp.stack(wgt_c, axis=-1))
    idx = jnp.stack(idx_bands, axis=1).reshape(4 * nbands, N, 4)
    wgt = jnp.stack(wgt_bands, axis=1).reshape(4 * nbands, N, 4)
    if npad > N:
        idx = jnp.pad(idx, ((0, 0), (0, npad - N), (0, 0)), constant_values=band_n)
        wgt = jnp.pad(wgt, ((0, 0), (0, npad - N), (0, 0)))
    nchunks = npad // _CHUNK
    idx = idx.reshape(4 * nbands, nchunks, 1, _CHUNK * 4)
    wgt = wgt.reshape(4 * nbands, nchunks, 1, _CHUNK * 4)
    return idx, wgt


def _scatter_loop(idx_ref, w_ref, acc_ref, vrow):
    def outer(oi, carry):
        for u in range(_UNROLL):
            i = oi * _UNROLL + u
            v = vrow(i)
            b = i * 4
            i0 = idx_ref[0, b]
            i1 = idx_ref[0, b + 1]
            i2 = idx_ref[0, b + 2]
            i3 = idx_ref[0, b + 3]
            w0 = w_ref[0, b]
            w1 = w_ref[0, b + 1]
            w2 = w_ref[0, b + 2]
            w3 = w_ref[0, b + 3]
            a0 = acc_ref[pl.ds(i0, 1), 0, :]
            a1 = acc_ref[pl.ds(i1, 1), 0, :]
            a2 = acc_ref[pl.ds(i2, 1), 0, :]
            a3 = acc_ref[pl.ds(i3, 1), 0, :]
            acc_ref[pl.ds(i0, 1), 0, :] = a0 + w0 * v
            acc_ref[pl.ds(i1, 1), 0, :] = a1 + w1 * v
            acc_ref[pl.ds(i2, 1), 0, :] = a2 + w2 * v
            acc_ref[pl.ds(i3, 1), 0, :] = a3 + w3 * v
        return carry
    jax.lax.fori_loop(0, _CHUNK // _UNROLL, outer, 0, unroll=1)


def _band_tile(band_n):
    for t in (512, 384, 256, 192, 128, 64, 32, 16, 8, 4, 2, 1):
        if band_n % t == 0:
            return t
    return 1


def _splat_body(x_ref, idx_ref, w_ref, o_ref, vscr, *,
                cin, nchunks, band_n, tile):
    c = pl.program_id(1)
    c_pad = o_ref.shape[-1]

    @pl.when(c == 0)
    def _():
        def z(t, carry):
            o_ref[pl.ds(t * tile, tile)] = jnp.zeros((tile, 1, c_pad), jnp.float32)
            return carry
        jax.lax.fori_loop(0, band_n // tile + 1, z, 0, unroll=1)

    x = x_ref[...]
    m = x[:, :, cin - 1:cin]
    e = jnp.exp(jnp.clip(-m, -20.0, 20.0))
    pad = c_pad - (cin + 2)
    vscr[...] = jnp.concatenate(
        [x * e, e, jnp.ones_like(e),
         jnp.zeros((x.shape[0], 1, pad), jnp.float32)], axis=2)

    _scatter_loop(idx_ref, w_ref, o_ref, lambda i: vscr[pl.ds(i, 1), 0, :])

    @pl.when(c == nchunks - 1)
    def _():
        def nrm(t, carry):
            y = o_ref[pl.ds(t * tile, tile)]
            den = y[:, :, cin:cin + 1] + EPS
            feat = y[:, :, :cin] / den
            a = y[:, :, cin + 1:cin + 2]
            msk = jnp.where(a / (a + EPS) <= 0.5, 1.0, 0.0)
            pad = c_pad - cin - 1
            o_ref[pl.ds(t * tile, tile)] = jnp.concatenate(
                [feat, msk, jnp.zeros((tile, 1, pad), jnp.float32)], axis=2)
            return carry
        jax.lax.fori_loop(0, band_n // tile, nrm, 0, unroll=1)


def _hole_body(idx_ref, w_ref, o_ref, *, nchunks, band_n, tile):
    c = pl.program_id(1)

    @pl.when(c == 0)
    def _():
        def z(t, carry):
            o_ref[pl.ds(t * tile, tile)] = jnp.zeros(
                (tile, 1, o_ref.shape[-1]), jnp.float32)
            return carry
        jax.lax.fori_loop(0, band_n // tile + 1, z, 0, unroll=1)

    _scatter_loop(idx_ref, w_ref, o_ref, lambda i: 1.0)

    @pl.when(c == nchunks - 1)
    def _():
        def nrm(t, carry):
            a = o_ref[pl.ds(t * tile, tile)]
            o_ref[pl.ds(t * tile, tile)] = jnp.where(
                a / (a + EPS) <= 0.5, 1.0, 0.0)
            return carry
        jax.lax.fori_loop(0, band_n // tile, nrm, 0, unroll=1)


def _splat_level(X, flow, nbands, name):
    _, _, H, W = flow.shape
    N = H * W
    cin = X.shape[-1]
    c_eff = cin + 2
    c_pad = ((c_eff + 127) // 128) * 128
    band_n = N // nbands
    tile = _band_tile(band_n)
    bandp = band_n + tile
    npad = ((N + _CHUNK - 1) // _CHUNK) * _CHUNK
    nchunks = npad // _CHUNK
    if npad > N:
        X = jnp.pad(X, ((0, 0), (0, npad - N), (0, 0), (0, 0)))
    idx, wgt = _corner_tables(flow, nbands, band_n, npad)
    P = 4 * nbands
    out = pl.pallas_call(
        functools.partial(_splat_body, cin=cin, nchunks=nchunks,
                          band_n=band_n, tile=tile),
        grid=(P, nchunks),
        in_specs=[
            pl.BlockSpec((None, _CHUNK, 1, cin),
                         lambda p, c: (p // nbands, c, 0, 0)),
            pl.BlockSpec((None, None, 1, _CHUNK * 4),
                         lambda p, c: (p, c, 0, 0),
                         memory_space=pltpu.SMEM),
            pl.BlockSpec((None, None, 1, _CHUNK * 4),
                         lambda p, c: (p, c, 0, 0),
                         memory_space=pltpu.SMEM),
        ],
        out_specs=pl.BlockSpec((None, bandp, 1, c_pad),
                               lambda p, c: (p, 0, 0, 0),
                               pipeline_mode=pl.Buffered(buffer_count=1)),
        out_shape=jax.ShapeDtypeStruct((P, bandp, 1, c_pad), jnp.float32),
        scratch_shapes=[
            pltpu.VMEM((_CHUNK, 1, c_pad), jnp.float32),
        ],
        compiler_params=pltpu.CompilerParams(
            dimension_semantics=("parallel", "arbitrary")),
        name=name,
    )(X, idx, wgt)
    res = out[:, :band_n, 0, :cin + 1]
    return res.reshape(4, nbands * band_n, cin + 1)


def _hole_level(flow, name):
    _, _, H, W = flow.shape
    N = H * W
    band_n = N
    tile = _band_tile(band_n)
    bandp = band_n + tile
    npad = ((N + _CHUNK - 1) // _CHUNK) * _CHUNK
    nchunks = npad // _CHUNK
    idx, wgt = _corner_tables(flow, 1, band_n, npad)
    out = pl.pallas_call(
        functools.partial(_hole_body, nchunks=nchunks,
                          band_n=band_n, tile=tile),
        grid=(4, nchunks),
        in_specs=[
            pl.BlockSpec((None, None, 1, _CHUNK * 4),
                         lambda p, c: (p, c, 0, 0),
                         memory_space=pltpu.SMEM),
            pl.BlockSpec((None, None, 1, _CHUNK * 4),
                         lambda p, c: (p, c, 0, 0),
                         memory_space=pltpu.SMEM),
        ],
        out_specs=pl.BlockSpec((None, bandp, 1, 128),
                               lambda p, c: (p, 0, 0, 0),
                               pipeline_mode=pl.Buffered(buffer_count=1)),
        out_shape=jax.ShapeDtypeStruct((4, bandp, 1, 128), jnp.float32),
        compiler_params=pltpu.CompilerParams(
            dimension_semantics=("parallel", "arbitrary")),
        name=name,
    )(idx, wgt)
    return out[:, :band_n, 0, :1]


def _pack_values(enc1, enc2, m1, m2):
    xa = jnp.concatenate([enc1, m1], 1)
    xb = jnp.concatenate([enc2, m2], 1)
    X = jnp.concatenate([xa, xb], 0)
    _, cin, H, W = X.shape
    return X.transpose(0, 2, 3, 1).reshape(4, H * W, 1, cin)


def _unpack(res, H, W):
    r = res.reshape(4, H, W, -1).transpose(0, 3, 1, 2)
    return r[:2], r[2:]


def kernel(enc1_0, enc1_1, enc1_2, enc2_0, enc2_1, enc2_2,
           metric1, metric2, flow_fwd, flow_bwd):
    enc1 = (enc1_0, enc1_1, enc1_2)
    enc2 = (enc2_0, enc2_1, enc2_2)
    m1, m2, ff, fb = metric1, metric2, flow_fwd, flow_bwd
    nbands_per_level = (3, 1, 1)
    outs = []
    masks = []
    H = W = h = w = None
    for lvl in range(3):
        one, two = enc1[lvl], enc2[lvl]
        H, W = one.shape[-2:]
        h, w = ff.shape[-2:]
        if lvl != 0:
            m1 = _resize(m1, H, W)
            m2 = _resize(m2, H, W)
            ff = _resize(ff, H, W) * (H / h)
            fb = _resize(fb, H, W) * (H / h)
        X = _pack_values(one, two, m1, m2)
        flow4 = jnp.concatenate([ff, fb], 0)
        res = _splat_level(X, flow4, nbands_per_level[lvl], f"splat_l{lvl}")
        cin = X.shape[-1]
        feat_f, feat_b = _unpack(res[:, :, :cin], H, W)
        mask_f, mask_b = _unpack(res[:, :, cin:cin + 1], H, W)
        outs.append((feat_f, feat_b))
        masks.append((mask_f, mask_b))
    ff = _resize(ff, H // 2, W // 2) * (H // 2) / h
    fb = _resize(fb, H // 2, W // 2) * (H // 2) / h
    flow4 = jnp.concatenate([ff, fb], 0)
    mres = _hole_level(flow4, "splat_hole_final")
    mask_f, mask_b = _unpack(mres, H // 2, W // 2)
    masks.append((mask_f, mask_b))
    return tuple(outs), tuple(masks)


# speedup vs baseline: 1.0067x; 1.0067x over previous
import functools

import jax
import jax.numpy as jnp
from jax.experimental import pallas as pl
from jax.experimental.pallas import tpu as pltpu

EPS = 1e-7
_CHUNK = 1024
_UNROLL = 4


def _src_idx(out_size, in_size):
    scale = in_size / out_size
    s = (jnp.arange(out_size, dtype=jnp.float32) + 0.5) * scale - 0.5
    s = jnp.maximum(s, 0.0)
    i0 = jnp.floor(s).astype(jnp.int32)
    i1 = jnp.minimum(i0 + 1, in_size - 1)
    lam = s - i0.astype(jnp.float32)
    return i0, i1, lam


def _resize(x, out_h, out_w):
    H, W = x.shape[-2:]
    y0, y1, ly = _src_idx(out_h, H)
    x0, x1, lx = _src_idx(out_w, W)
    r = x[:, :, y0, :] * (1.0 - ly)[None, None, :, None] + x[:, :, y1, :] * ly[None, None, :, None]
    return r[:, :, :, x0] * (1.0 - lx)[None, None, None, :] + r[:, :, :, x1] * lx[None, None, None, :]


def _corner_tables(flow, nbands, band_n, npad):
    _, _, H, W = flow.shape
    N = H * W
    dump = jnp.int32(band_n)
    fx = jnp.arange(W, dtype=jnp.float32)[None, None, :] + flow[:, 0]
    fy = jnp.arange(H, dtype=jnp.float32)[None, :, None] + flow[:, 1]
    fx = jnp.clip(fx, -4.0, W + 4.0)
    fy = jnp.clip(fy, -4.0, H + 4.0)
    x0f = jnp.floor(fx)
    y0f = jnp.floor(fy)
    wx1 = fx - x0f
    wy1 = fy - y0f
    x0 = x0f.astype(jnp.int32)
    y0 = y0f.astype(jnp.int32)
    idx_bands = []
    wgt_bands = []
    for b in range(nbands):
        lo = b * band_n
        idx_c = []
        wgt_c = []
        for dy in (0, 1):
            for dx in (0, 1):
                xk = x0 + dx
                yk = y0 + dy
                wk = (wx1 if dx else 1.0 - wx1) * (wy1 if dy else 1.0 - wy1)
                local = yk * W + xk - lo
                ok = (xk >= 0) & (xk < W) & (yk >= 0) & (yk < H) \
                    & (local >= 0) & (local < band_n)
                idx_c.append(jnp.where(ok, local, dump).reshape(4, N))
                wgt_c.append(jnp.where(ok, wk, 0.0).reshape(4, N))
        idx_bands.append(jnp.stack(idx_c, axis=-1))
        wgt_bands.append(jnp.stack(wgt_c, axis=-1))
    idx = jnp.stack(idx_bands, axis=1).reshape(4 * nbands, N, 4)
    wgt = jnp.stack(wgt_bands, axis=1).reshape(4 * nbands, N, 4)
    if npad > N:
        idx = jnp.pad(idx, ((0, 0), (0, npad - N), (0, 0)), constant_values=band_n)
        wgt = jnp.pad(wgt, ((0, 0), (0, npad - N), (0, 0)))
    nchunks = npad // _CHUNK
    idx = idx.reshape(4 * nbands, nchunks, 1, _CHUNK * 4)
    wgt = wgt.reshape(4 * nbands, nchunks, 1, _CHUNK * 4)
    return idx, wgt


def _scatter_loop(idx_ref, w_ref, acc_ref, vrow):
    def outer(oi, carry):
        for u in range(_UNROLL):
            i = oi * _UNROLL + u
            v = vrow(i)
            b = i * 4
            i0 = idx_ref[0, b]
            i1 = idx_ref[0, b + 1]
            i2 = idx_ref[0, b + 2]
            i3 = idx_ref[0, b + 3]
            w0 = w_ref[0, b]
            w1 = w_ref[0, b + 1]
            w2 = w_ref[0, b + 2]
            w3 = w_ref[0, b + 3]
            a0 = acc_ref[pl.ds(i0, 1), 0, :]
            a1 = acc_ref[pl.ds(i1, 1), 0, :]
            a2 = acc_ref[pl.ds(i2, 1), 0, :]
            a3 = acc_ref[pl.ds(i3, 1), 0, :]
            acc_ref[pl.ds(i0, 1), 0, :] = a0 + w0 * v
            acc_ref[pl.ds(i1, 1), 0, :] = a1 + w1 * v
            acc_ref[pl.ds(i2, 1), 0, :] = a2 + w2 * v
            acc_ref[pl.ds(i3, 1), 0, :] = a3 + w3 * v
        return carry
    jax.lax.fori_loop(0, _CHUNK // _UNROLL, outer, 0, unroll=1)


def _band_tile(band_n):
    for t in (512, 384, 256, 192, 128, 64, 32, 16, 8, 4, 2, 1):
        if band_n % t == 0:
            return t
    return 1


def _splat_body(x_ref, idx_ref, w_ref, o_ref, vscr, *,
                cin, nchunks, band_n, tile):
    c = pl.program_id(1)
    c_pad = o_ref.shape[-1]

    @pl.when(c == 0)
    def _():
        def z(t, carry):
            o_ref[pl.ds(t * tile, tile)] = jnp.zeros((tile, 1, c_pad), jnp.float32)
            return carry
        jax.lax.fori_loop(0, band_n // tile + 1, z, 0, unroll=1)

    x = x_ref[...]
    m = x[:, :, cin - 1:cin]
    e = jnp.exp(jnp.clip(-m, -20.0, 20.0))
    pad = c_pad - (cin + 2)
    vscr[...] = jnp.concatenate(
        [x * e, e, jnp.ones_like(e),
         jnp.zeros((x.shape[0], 1, pad), jnp.float32)], axis=2)

    _scatter_loop(idx_ref, w_ref, o_ref, lambda i: vscr[pl.ds(i, 1), 0, :])

    @pl.when(c == nchunks - 1)
    def _():
        def nrm(t, carry):
            y = o_ref[pl.ds(t * tile, tile)]
            den = y[:, :, cin:cin + 1] + EPS
            feat = y[:, :, :cin] / den
            a = y[:, :, cin + 1:cin + 2]
            msk = jnp.where(a / (a + EPS) <= 0.5, 1.0, 0.0)
            pad = c_pad - cin - 1
            o_ref[pl.ds(t * tile, tile)] = jnp.concatenate(
                [feat, msk, jnp.zeros((tile, 1, pad), jnp.float32)], axis=2)
            return carry
        jax.lax.fori_loop(0, band_n // tile, nrm, 0, unroll=1)


def _hole_body(idx_ref, w_ref, o_ref, *, nchunks, band_n, tile):
    c = pl.program_id(1)

    @pl.when(c == 0)
    def _():
        def z(t, carry):
            o_ref[pl.ds(t * tile, tile)] = jnp.zeros(
                (tile, 1, o_ref.shape[-1]), jnp.float32)
            return carry
        jax.lax.fori_loop(0, band_n // tile + 1, z, 0, unroll=1)

    _scatter_loop(idx_ref, w_ref, o_ref, lambda i: 1.0)

    @pl.when(c == nchunks - 1)
    def _():
        def nrm(t, carry):
            a = o_ref[pl.ds(t * tile, tile)]
            o_ref[pl.ds(t * tile, tile)] = jnp.where(
                a / (a + EPS) <= 0.5, 1.0, 0.0)
            return carry
        jax.lax.fori_loop(0, band_n // tile, nrm, 0, unroll=1)


def _splat_level(X, flow, nbands, name):
    _, _, H, W = flow.shape
    N = H * W
    cin = X.shape[-1]
    c_eff = cin + 2
    c_pad = ((c_eff + 127) // 128) * 128
    band_n = N // nbands
    tile = _band_tile(band_n)
    bandp = band_n + tile
    npad = ((N + _CHUNK - 1) // _CHUNK) * _CHUNK
    nchunks = npad // _CHUNK
    if npad > N:
        X = jnp.pad(X, ((0, 0), (0, npad - N), (0, 0), (0, 0)))
    idx, wgt = _corner_tables(flow, nbands, band_n, npad)
    P = 4 * nbands
    out = pl.pallas_call(
        functools.partial(_splat_body, cin=cin, nchunks=nchunks,
                          band_n=band_n, tile=tile),
        grid=(P, nchunks),
        in_specs=[
            pl.BlockSpec((None, _CHUNK, 1, cin),
                         lambda p, c: (p // nbands, c, 0, 0)),
            pl.BlockSpec((None, None, 1, _CHUNK * 4),
                         lambda p, c: (p, c, 0, 0),
                         memory_space=pltpu.SMEM),
            pl.BlockSpec((None, None, 1, _CHUNK * 4),
                         lambda p, c: (p, c, 0, 0),
                         memory_space=pltpu.SMEM),
        ],
        out_specs=pl.BlockSpec((None, bandp, 1, c_pad),
                               lambda p, c: (p, 0, 0, 0),
                               pipeline_mode=pl.Buffered(buffer_count=1)),
        out_shape=jax.ShapeDtypeStruct((P, bandp, 1, c_pad), jnp.float32),
        scratch_shapes=[
            pltpu.VMEM((_CHUNK, 1, c_pad), jnp.float32),
        ],
        compiler_params=pltpu.CompilerParams(
            dimension_semantics=("parallel", "arbitrary"),
            disable_bounds_checks=True),
        name=name,
    )(X, idx, wgt)
    res = out[:, :band_n, 0, :cin + 1]
    return res.reshape(4, nbands * band_n, cin + 1)


def _hole_level(flow, name):
    _, _, H, W = flow.shape
    N = H * W
    band_n = N
    tile = _band_tile(band_n)
    bandp = band_n + tile
    npad = ((N + _CHUNK - 1) // _CHUNK) * _CHUNK
    nchunks = npad // _CHUNK
    idx, wgt = _corner_tables(flow, 1, band_n, npad)
    out = pl.pallas_call(
        functools.partial(_hole_body, nchunks=nchunks,
                          band_n=band_n, tile=tile),
        grid=(4, nchunks),
        in_specs=[
            pl.BlockSpec((None, None, 1, _CHUNK * 4),
                         lambda p, c: (p, c, 0, 0),
                         memory_space=pltpu.SMEM),
            pl.BlockSpec((None, None, 1, _CHUNK * 4),
                         lambda p, c: (p, c, 0, 0),
                         memory_space=pltpu.SMEM),
        ],
        out_specs=pl.BlockSpec((None, bandp, 1, 128),
                               lambda p, c: (p, 0, 0, 0),
                               pipeline_mode=pl.Buffered(buffer_count=1)),
        out_shape=jax.ShapeDtypeStruct((4, bandp, 1, 128), jnp.float32),
        compiler_params=pltpu.CompilerParams(
            dimension_semantics=("parallel", "arbitrary"),
            disable_bounds_checks=True),
        name=name,
    )(idx, wgt)
    return out[:, :band_n, 0, :1]


def _pack_values(enc1, enc2, m1, m2):
    xa = jnp.concatenate([enc1, m1], 1)
    xb = jnp.concatenate([enc2, m2], 1)
    X = jnp.concatenate([xa, xb], 0)
    _, cin, H, W = X.shape
    return X.transpose(0, 2, 3, 1).reshape(4, H * W, 1, cin)


def _unpack(res, H, W):
    r = res.reshape(4, H, W, -1).transpose(0, 3, 1, 2)
    return r[:2], r[2:]


def kernel(enc1_0, enc1_1, enc1_2, enc2_0, enc2_1, enc2_2,
           metric1, metric2, flow_fwd, flow_bwd):
    enc1 = (enc1_0, enc1_1, enc1_2)
    enc2 = (enc2_0, enc2_1, enc2_2)
    m1, m2, ff, fb = metric1, metric2, flow_fwd, flow_bwd
    nbands_per_level = (3, 1, 1)
    outs = []
    masks = []
    H = W = h = w = None
    for lvl in range(3):
        one, two = enc1[lvl], enc2[lvl]
        H, W = one.shape[-2:]
        h, w = ff.shape[-2:]
        if lvl != 0:
            m1 = _resize(m1, H, W)
            m2 = _resize(m2, H, W)
            ff = _resize(ff, H, W) * (H / h)
            fb = _resize(fb, H, W) * (H / h)
        X = _pack_values(one, two, m1, m2)
        flow4 = jnp.concatenate([ff, fb], 0)
        res = _splat_level(X, flow4, nbands_per_level[lvl], f"splat_l{lvl}")
        cin = X.shape[-1]
        feat_f, feat_b = _unpack(res[:, :, :cin], H, W)
        mask_f, mask_b = _unpack(res[:, :, cin:cin + 1], H, W)
        outs.append((feat_f, feat_b))
        masks.append((mask_f, mask_b))
    ff = _resize(ff, H // 2, W // 2) * (H // 2) / h
    fb = _resize(fb, H // 2, W // 2) * (H // 2) / h
    flow4 = jnp.concatenate([ff, fb], 0)
    mres = _hole_level(flow4, "splat_hole_final")
    mask_f, mask_b = _unpack(mres, H // 2, W // 2)
    masks.append((mask_f, mask_b))
    return tuple(outs), tuple(masks)
